# baseline (speedup 1.0000x reference)
import functools

import jax
import jax.numpy as jnp
import numpy as np
from jax import lax
from jax.experimental import pallas as pl
from jax.experimental.pallas import tpu as pltpu
from jax.experimental.pallas import tpu_sc as plsc

_EPS = float(np.finfo(np.float64).eps)


def _gating_body(x_ref, wg_ref, i1_ref, i2_ref, g1_ref, g2_ref, cnt_ref,
                 acc_ref):
    i = pl.program_id(0)
    logits = lax.dot_general(x_ref[...], wg_ref[...], (((1,), (0,)), ((), ())),
                             preferred_element_type=jnp.float32)
    e = logits.shape[1]
    col = lax.broadcasted_iota(jnp.int32, logits.shape, 1)
    m1 = jnp.max(logits, axis=1, keepdims=True)
    i1 = jnp.min(jnp.where(logits == m1, col, e), axis=1, keepdims=True)
    masked = jnp.where(col == i1, -jnp.inf, logits)
    m2 = jnp.max(masked, axis=1, keepdims=True)
    i2 = jnp.min(jnp.where(masked == m2, col, e), axis=1, keepdims=True)
    t = jnp.exp(m2 - m1)
    denom = 1.0 + t
    i1_ref[...] = i1
    i2_ref[...] = i2
    g1_ref[...] = 1.0 / denom
    g2_ref[...] = t / denom
    hist = (jnp.sum((i1 == col[: i1.shape[0]]).astype(jnp.int32), axis=0,
                    keepdims=True)
            + jnp.sum((i2 == col[: i2.shape[0]]).astype(jnp.int32), axis=0,
                      keepdims=True))

    @pl.when(i == 0)
    def _():
        acc_ref[...] = jnp.zeros_like(acc_ref)

    acc_ref[0:1, :] += hist

    @pl.when(i == pl.num_programs(0) - 1)
    def _():
        cnt_ref[...] = acc_ref[0:1, :]


def _routing_body(i1_ref, i2_ref, poffs_ref, p0_ref, p1_ref, acc_ref, lt_ref):
    i = pl.program_id(0)
    tr = i1_ref.shape[0]
    ee = jnp.concatenate([i1_ref[...], i2_ref[...]], axis=0)
    e = poffs_ref.shape[1] - 1
    col = lax.broadcasted_iota(jnp.int32, (2 * tr, e), 1)
    onehot_b = ee == col
    onehot_f = onehot_b.astype(jnp.float32)

    @pl.when(i == 0)
    def _():
        acc_ref[...] = jnp.zeros_like(acc_ref)
        r = lax.broadcasted_iota(jnp.int32, (2 * tr, 2 * tr), 0)
        c = lax.broadcasted_iota(jnp.int32, (2 * tr, 2 * tr), 1)
        lt_ref[...] = (c < r).astype(jnp.float32)

    excl = lax.dot_general(lt_ref[...], onehot_f, (((1,), (0,)), ((), ())),
                           preferred_element_type=jnp.float32)
    rank = jnp.sum(excl * onehot_f, axis=1, keepdims=True).astype(jnp.int32)
    base = jnp.sum(jnp.where(onehot_b, poffs_ref[0:1, :e], 0), axis=1,
                   keepdims=True)
    run = jnp.sum(jnp.where(onehot_b, acc_ref[0:1, :], 0), axis=1,
                  keepdims=True)
    slot = base + run + rank
    p0_ref[...] = slot[:tr]
    p1_ref[...] = slot[tr:]
    acc_ref[0:1, :] += jnp.sum(onehot_b.astype(jnp.int32), axis=0,
                               keepdims=True)


def _gmm_body(te_ref, na_ref, xs_ref, w1_ref, b1_ref, w2_ref, b2_ref, ys_ref):
    i = pl.program_id(0)

    @pl.when(i < na_ref[0])
    def _():
        h = lax.dot_general(xs_ref[...], w1_ref[0], (((1,), (0,)), ((), ())),
                            preferred_element_type=jnp.float32)
        h = jnp.maximum(h + b1_ref[0], 0.0)
        o = lax.dot_general(h, w2_ref[0], (((1,), (0,)), ((), ())),
                            preferred_element_type=jnp.float32)
        ys_ref[...] = jnp.exp(jnp.tanh(o + b2_ref[0]) * 10.0)


def _combine_body(y1_ref, y2_ref, g1_ref, g2_ref, o_ref):
    s = g1_ref[...] * y1_ref[...] + g2_ref[...] * y2_ref[...]
    o_ref[...] = jnp.log(jnp.where(s == 0.0, _EPS, s))


def _sc_scatter_rows(x, pos0, pos1, p):
    n, d = x.shape
    nw = 32
    ch = 128
    t_per_w = n // nw
    nch = t_per_w // ch
    mesh = plsc.VectorSubcoreMesh(core_axis_name="c", subcore_axis_name="s")

    @functools.partial(
        pl.kernel,
        mesh=mesh,
        out_type=jax.ShapeDtypeStruct((p, d), jnp.float32),
        scratch_types=[
            pltpu.VMEM((2, ch), jnp.int32),
            pltpu.VMEM((ch, d), jnp.float32),
            pltpu.SemaphoreType.DMA,
        ],
    )
    def scatter_k(x_hbm, p0_hbm, p1_hbm, xs_hbm, posb, rows, sem):
        wid = lax.axis_index("s") * 2 + lax.axis_index("c")
        for cc in range(nch):
            off = base = wid * t_per_w + cc * ch
            li0 = pltpu.async_copy(p0_hbm.at[pl.ds(off, ch)], posb.at[0], sem)
            li1 = pltpu.async_copy(p1_hbm.at[pl.ds(off, ch)], posb.at[1], sem)
            lr = pltpu.async_copy(x_hbm.at[pl.ds(off, ch)], rows, sem)
            li0.wait()
            li1.wait()
            lr.wait()
            cp0 = pltpu.async_copy(rows, xs_hbm.at[posb.at[0]], sem)
            cp1 = pltpu.async_copy(rows, xs_hbm.at[posb.at[1]], sem)
            cp0.wait()
            cp1.wait()

    return scatter_k(x, pos0, pos1)


def _sc_gather(table, idx):
    _, d = table.shape
    b = idx.shape[0]
    nw = 32
    ch = 128
    b_per_w = b // nw
    nch = b_per_w // ch
    mesh = plsc.VectorSubcoreMesh(core_axis_name="c", subcore_axis_name="s")

    @functools.partial(
        pl.kernel,
        mesh=mesh,
        out_type=jax.ShapeDtypeStruct((b, d), jnp.float32),
        scratch_types=[
            pltpu.VMEM((nch, ch), jnp.int32),
            pltpu.VMEM((ch, d), jnp.float32),
            pltpu.SemaphoreType.DMA,
        ],
    )
    def gather_k(table_hbm, idx_hbm, out_hbm, idxb, rows, sem):
        wid = lax.axis_index("s") * 2 + lax.axis_index("c")
        base = wid * b_per_w
        for j in range(nch):
            pltpu.sync_copy(idx_hbm.at[pl.ds(base + j * ch, ch)], idxb.at[j])
        for j in range(nch):
            pltpu.async_copy(table_hbm.at[idxb.at[j]], rows, sem).wait()
            pltpu.sync_copy(rows, out_hbm.at[pl.ds(base + j * ch, ch)])

    return gather_k(table, idx)


def kernel(x, w_gate, W1, b1, W2, b2):
    n, d = x.shape
    e = w_gate.shape[1]
    h = W1.shape[2]
    out_d = W2.shape[2]
    tm = 192
    p = 2 * n + (tm - 1) * min(e, 2 * n)
    p = ((p + tm - 1) // tm) * tm
    tn = p // tm

    tg = 512
    i1, i2, g1, g2, counts = pl.pallas_call(
        _gating_body,
        grid=(n // tg,),
        in_specs=[
            pl.BlockSpec((tg, d), lambda i: (i, 0)),
            pl.BlockSpec((d, e), lambda i: (0, 0)),
        ],
        out_specs=[
            pl.BlockSpec((tg, 1), lambda i: (i, 0)),
            pl.BlockSpec((tg, 1), lambda i: (i, 0)),
            pl.BlockSpec((tg, 1), lambda i: (i, 0)),
            pl.BlockSpec((tg, 1), lambda i: (i, 0)),
            pl.BlockSpec((1, e), lambda i: (0, 0)),
        ],
        out_shape=[
            jax.ShapeDtypeStruct((n, 1), jnp.int32),
            jax.ShapeDtypeStruct((n, 1), jnp.int32),
            jax.ShapeDtypeStruct((n, 1), jnp.float32),
            jax.ShapeDtypeStruct((n, 1), jnp.float32),
            jax.ShapeDtypeStruct((1, e), jnp.int32),
        ],
        scratch_shapes=[pltpu.VMEM((8, e), jnp.int32)],
    )(x, w_gate)

    pcounts = ((counts[0] + tm - 1) // tm) * tm
    poffs = jnp.concatenate(
        [jnp.zeros((1,), jnp.int32), jnp.cumsum(pcounts).astype(jnp.int32)])
    tile_start = jnp.arange(tn, dtype=jnp.int32) * tm
    tile_expert = jnp.minimum(
        jnp.sum((tile_start[:, None] >= poffs[None, 1:]).astype(jnp.int32),
                axis=1), e - 1).astype(jnp.int32)
    n_active = (poffs[e] // tm).reshape(1)

    tr = 512
    pos0, pos1 = pl.pallas_call(
        _routing_body,
        grid=(n // tr,),
        in_specs=[
            pl.BlockSpec((tr, 1), lambda i: (i, 0)),
            pl.BlockSpec((tr, 1), lambda i: (i, 0)),
            pl.BlockSpec((1, e + 1), lambda i: (0, 0)),
        ],
        out_specs=[
            pl.BlockSpec((tr, 1), lambda i: (i, 0)),
            pl.BlockSpec((tr, 1), lambda i: (i, 0)),
        ],
        out_shape=[
            jax.ShapeDtypeStruct((n, 1), jnp.int32),
            jax.ShapeDtypeStruct((n, 1), jnp.int32),
        ],
        scratch_shapes=[pltpu.VMEM((8, e), jnp.int32),
                        pltpu.VMEM((2 * tr, 2 * tr), jnp.float32)],
    )(i1, i2, poffs.reshape(1, e + 1))
    pos0 = pos0.reshape(n)
    pos1 = pos1.reshape(n)

    xs = _sc_scatter_rows(x, pos0, pos1, p)

    grid_spec = pltpu.PrefetchScalarGridSpec(
        num_scalar_prefetch=2,
        grid=(tn,),
        in_specs=[
            pl.BlockSpec((tm, d),
                         lambda i, te, na: (jnp.minimum(i, na[0] - 1), 0)),
            pl.BlockSpec((1, d, h), lambda i, te, na: (te[i], 0, 0)),
            pl.BlockSpec((1, 1, h), lambda i, te, na: (te[i], 0, 0)),
            pl.BlockSpec((1, h, out_d), lambda i, te, na: (te[i], 0, 0)),
            pl.BlockSpec((1, 1, out_d), lambda i, te, na: (te[i], 0, 0)),
        ],
        out_specs=pl.BlockSpec(
            (tm, out_d), lambda i, te, na: (jnp.minimum(i, na[0] - 1), 0)),
    )
    ys = pl.pallas_call(
        _gmm_body,
        grid_spec=grid_spec,
        out_shape=jax.ShapeDtypeStruct((p, out_d), jnp.float32),
    )(tile_expert, n_active, xs, W1, b1.reshape(e, 1, h), W2,
      b2.reshape(e, 1, out_d))

    tb = 512
    nh = 2
    cn = n // nh
    outs = []
    for hh in range(nh):
        s = hh * cn
        idx_h = jnp.concatenate(
            [lax.slice_in_dim(pos0, s, s + cn), lax.slice_in_dim(pos1, s, s + cn)])
        yt_h = _sc_gather(ys, idx_h)
        out_h = pl.pallas_call(
            _combine_body,
            grid=(cn // tb,),
            in_specs=[
                pl.BlockSpec((tb, out_d), lambda i: (i, 0)),
                pl.BlockSpec((tb, out_d), lambda i, c=cn // tb: (i + c, 0)),
                pl.BlockSpec((tb, 1), lambda i: (i, 0)),
                pl.BlockSpec((tb, 1), lambda i: (i, 0)),
            ],
            out_specs=pl.BlockSpec((tb, out_d), lambda i: (i, 0)),
            out_shape=jax.ShapeDtypeStruct((cn, out_d), jnp.float32),
        )(yt_h, yt_h, lax.slice_in_dim(g1, s, s + cn),
          lax.slice_in_dim(g2, s, s + cn))
        outs.append(out_h)
    return jnp.concatenate(outs, axis=0)


# speedup vs baseline: 1.0634x; 1.0634x over previous
import functools

import jax
import jax.numpy as jnp
import numpy as np
from jax import lax
from jax.experimental import pallas as pl
from jax.experimental.pallas import tpu as pltpu
from jax.experimental.pallas import tpu_sc as plsc

_EPS = float(np.finfo(np.float64).eps)


def _gating_body(x_ref, wg_ref, i1_ref, i2_ref, g1_ref, g2_ref, cnt_ref,
                 acc_ref):
    i = pl.program_id(0)
    logits = lax.dot_general(x_ref[...], wg_ref[...], (((1,), (0,)), ((), ())),
                             preferred_element_type=jnp.float32)
    e = logits.shape[1]
    col = lax.broadcasted_iota(jnp.int32, logits.shape, 1)
    m1 = jnp.max(logits, axis=1, keepdims=True)
    i1 = jnp.min(jnp.where(logits == m1, col, e), axis=1, keepdims=True)
    masked = jnp.where(col == i1, -jnp.inf, logits)
    m2 = jnp.max(masked, axis=1, keepdims=True)
    i2 = jnp.min(jnp.where(masked == m2, col, e), axis=1, keepdims=True)
    t = jnp.exp(m2 - m1)
    denom = 1.0 + t
    i1_ref[...] = i1
    i2_ref[...] = i2
    g1_ref[...] = 1.0 / denom
    g2_ref[...] = t / denom
    hist = (jnp.sum((i1 == col[: i1.shape[0]]).astype(jnp.int32), axis=0,
                    keepdims=True)
            + jnp.sum((i2 == col[: i2.shape[0]]).astype(jnp.int32), axis=0,
                      keepdims=True))

    @pl.when(i == 0)
    def _():
        acc_ref[...] = jnp.zeros_like(acc_ref)

    acc_ref[0:1, :] += hist

    @pl.when(i == pl.num_programs(0) - 1)
    def _():
        cnt_ref[...] = acc_ref[0:1, :]


def _routing_body(i1_ref, i2_ref, poffs_ref, p0_ref, p1_ref, acc_ref, lt_ref):
    i = pl.program_id(0)
    tr = i1_ref.shape[0]
    ee = jnp.concatenate([i1_ref[...], i2_ref[...]], axis=0)
    e = poffs_ref.shape[1] - 1
    col = lax.broadcasted_iota(jnp.int32, (2 * tr, e), 1)
    onehot_b = ee == col
    onehot_f = onehot_b.astype(jnp.float32)

    @pl.when(i == 0)
    def _():
        acc_ref[...] = jnp.zeros_like(acc_ref)
        r = lax.broadcasted_iota(jnp.int32, (2 * tr, 2 * tr), 0)
        c = lax.broadcasted_iota(jnp.int32, (2 * tr, 2 * tr), 1)
        lt_ref[...] = (c < r).astype(jnp.float32)

    excl = lax.dot_general(lt_ref[...], onehot_f, (((1,), (0,)), ((), ())),
                           preferred_element_type=jnp.float32)
    rank = jnp.sum(excl * onehot_f, axis=1, keepdims=True).astype(jnp.int32)
    base = jnp.sum(jnp.where(onehot_b, poffs_ref[0:1, :e], 0), axis=1,
                   keepdims=True)
    run = jnp.sum(jnp.where(onehot_b, acc_ref[0:1, :], 0), axis=1,
                  keepdims=True)
    slot = base + run + rank
    p0_ref[...] = slot[:tr]
    p1_ref[...] = slot[tr:]
    acc_ref[0:1, :] += jnp.sum(onehot_b.astype(jnp.int32), axis=0,
                               keepdims=True)


def _gmm_body(te_ref, na_ref, xs_ref, w1_ref, b1_ref, w2_ref, b2_ref, ys_ref):
    i = pl.program_id(0)

    @pl.when(i < na_ref[0])
    def _():
        h = lax.dot_general(xs_ref[...], w1_ref[0], (((1,), (0,)), ((), ())),
                            preferred_element_type=jnp.float32)
        h = jnp.maximum(h + b1_ref[0], 0.0)
        o = lax.dot_general(h, w2_ref[0], (((1,), (0,)), ((), ())),
                            preferred_element_type=jnp.float32)
        ys_ref[...] = jnp.exp(jnp.tanh(o + b2_ref[0]) * 10.0)


def _combine_body(y1_ref, y2_ref, g1_ref, g2_ref, o_ref):
    s = g1_ref[...] * y1_ref[...] + g2_ref[...] * y2_ref[...]
    o_ref[...] = jnp.log(jnp.where(s == 0.0, _EPS, s))


def _sc_scatter_rows(x, pos0, pos1, p):
    n, d = x.shape
    nw = 32
    ch = 128
    t_per_w = n // nw
    nch = t_per_w // ch
    mesh = plsc.VectorSubcoreMesh(core_axis_name="c", subcore_axis_name="s")

    @functools.partial(
        pl.kernel,
        mesh=mesh,
        out_type=jax.ShapeDtypeStruct((p, d), jnp.float32),
        scratch_types=[
            pltpu.VMEM((2, ch), jnp.int32),
            pltpu.VMEM((ch, d), jnp.float32),
            pltpu.SemaphoreType.DMA,
        ],
    )
    def scatter_k(x_hbm, p0_hbm, p1_hbm, xs_hbm, posb, rows, sem):
        wid = lax.axis_index("s") * 2 + lax.axis_index("c")
        for cc in range(nch):
            off = base = wid * t_per_w + cc * ch
            li0 = pltpu.async_copy(p0_hbm.at[pl.ds(off, ch)], posb.at[0], sem)
            li1 = pltpu.async_copy(p1_hbm.at[pl.ds(off, ch)], posb.at[1], sem)
            lr = pltpu.async_copy(x_hbm.at[pl.ds(off, ch)], rows, sem)
            li0.wait()
            li1.wait()
            lr.wait()
            cp0 = pltpu.async_copy(rows, xs_hbm.at[posb.at[0]], sem)
            cp1 = pltpu.async_copy(rows, xs_hbm.at[posb.at[1]], sem)
            cp0.wait()
            cp1.wait()

    return scatter_k(x, pos0, pos1)


def _sc_gather(table, idx):
    _, d = table.shape
    b = idx.shape[0]
    nw = 32
    ch = 128
    b_per_w = b // nw
    nch = b_per_w // ch
    mesh = plsc.VectorSubcoreMesh(core_axis_name="c", subcore_axis_name="s")

    @functools.partial(
        pl.kernel,
        mesh=mesh,
        out_type=jax.ShapeDtypeStruct((b, d), jnp.float32),
        scratch_types=[
            pltpu.VMEM((nch, ch), jnp.int32),
            pltpu.VMEM((ch, d), jnp.float32),
            pltpu.SemaphoreType.DMA,
        ],
    )
    def gather_k(table_hbm, idx_hbm, out_hbm, idxb, rows, sem):
        wid = lax.axis_index("s") * 2 + lax.axis_index("c")
        base = wid * b_per_w
        for j in range(nch):
            pltpu.sync_copy(idx_hbm.at[pl.ds(base + j * ch, ch)], idxb.at[j])
        for j in range(nch):
            pltpu.async_copy(table_hbm.at[idxb.at[j]], rows, sem).wait()
            pltpu.sync_copy(rows, out_hbm.at[pl.ds(base + j * ch, ch)])

    return gather_k(table, idx)


def kernel(x, w_gate, W1, b1, W2, b2):
    n, d = x.shape
    e = w_gate.shape[1]
    h = W1.shape[2]
    out_d = W2.shape[2]
    tm = 192
    p = 2 * n + (tm - 1) * min(e, 2 * n)
    p = ((p + tm - 1) // tm) * tm
    tn = p // tm

    tg = 1024
    i1, i2, g1, g2, counts = pl.pallas_call(
        _gating_body,
        grid=(n // tg,),
        in_specs=[
            pl.BlockSpec((tg, d), lambda i: (i, 0)),
            pl.BlockSpec((d, e), lambda i: (0, 0)),
        ],
        out_specs=[
            pl.BlockSpec((tg, 1), lambda i: (i, 0)),
            pl.BlockSpec((tg, 1), lambda i: (i, 0)),
            pl.BlockSpec((tg, 1), lambda i: (i, 0)),
            pl.BlockSpec((tg, 1), lambda i: (i, 0)),
            pl.BlockSpec((1, e), lambda i: (0, 0)),
        ],
        out_shape=[
            jax.ShapeDtypeStruct((n, 1), jnp.int32),
            jax.ShapeDtypeStruct((n, 1), jnp.int32),
            jax.ShapeDtypeStruct((n, 1), jnp.float32),
            jax.ShapeDtypeStruct((n, 1), jnp.float32),
            jax.ShapeDtypeStruct((1, e), jnp.int32),
        ],
        scratch_shapes=[pltpu.VMEM((8, e), jnp.int32)],
    )(x, w_gate)

    pcounts = ((counts[0] + tm - 1) // tm) * tm
    poffs = jnp.concatenate(
        [jnp.zeros((1,), jnp.int32), jnp.cumsum(pcounts).astype(jnp.int32)])
    tile_start = jnp.arange(tn, dtype=jnp.int32) * tm
    tile_expert = jnp.minimum(
        jnp.sum((tile_start[:, None] >= poffs[None, 1:]).astype(jnp.int32),
                axis=1), e - 1).astype(jnp.int32)
    n_active = (poffs[e] // tm).reshape(1)

    tr = 512
    pos0, pos1 = pl.pallas_call(
        _routing_body,
        grid=(n // tr,),
        in_specs=[
            pl.BlockSpec((tr, 1), lambda i: (i, 0)),
            pl.BlockSpec((tr, 1), lambda i: (i, 0)),
            pl.BlockSpec((1, e + 1), lambda i: (0, 0)),
        ],
        out_specs=[
            pl.BlockSpec((tr, 1), lambda i: (i, 0)),
            pl.BlockSpec((tr, 1), lambda i: (i, 0)),
        ],
        out_shape=[
            jax.ShapeDtypeStruct((n, 1), jnp.int32),
            jax.ShapeDtypeStruct((n, 1), jnp.int32),
        ],
        scratch_shapes=[pltpu.VMEM((8, e), jnp.int32),
                        pltpu.VMEM((2 * tr, 2 * tr), jnp.float32)],
    )(i1, i2, poffs.reshape(1, e + 1))
    pos0 = pos0.reshape(n)
    pos1 = pos1.reshape(n)

    xs = _sc_scatter_rows(x, pos0, pos1, p)

    grid_spec = pltpu.PrefetchScalarGridSpec(
        num_scalar_prefetch=2,
        grid=(tn,),
        in_specs=[
            pl.BlockSpec((tm, d),
                         lambda i, te, na: (jnp.minimum(i, na[0] - 1), 0)),
            pl.BlockSpec((1, d, h), lambda i, te, na: (te[i], 0, 0)),
            pl.BlockSpec((1, 1, h), lambda i, te, na: (te[i], 0, 0)),
            pl.BlockSpec((1, h, out_d), lambda i, te, na: (te[i], 0, 0)),
            pl.BlockSpec((1, 1, out_d), lambda i, te, na: (te[i], 0, 0)),
        ],
        out_specs=pl.BlockSpec(
            (tm, out_d), lambda i, te, na: (jnp.minimum(i, na[0] - 1), 0)),
    )
    ys = pl.pallas_call(
        _gmm_body,
        grid_spec=grid_spec,
        out_shape=jax.ShapeDtypeStruct((p, out_d), jnp.float32),
    )(tile_expert, n_active, xs, W1, b1.reshape(e, 1, h), W2,
      b2.reshape(e, 1, out_d))

    yt = _sc_gather(ys, jnp.concatenate([pos0, pos1]))

    tb = 512
    out = pl.pallas_call(
        _combine_body,
        grid=(n // tb,),
        in_specs=[
            pl.BlockSpec((tb, out_d), lambda i: (i, 0)),
            pl.BlockSpec((tb, out_d), lambda i: (i + n // tb, 0)),
            pl.BlockSpec((tb, 1), lambda i: (i, 0)),
            pl.BlockSpec((tb, 1), lambda i: (i, 0)),
        ],
        out_specs=pl.BlockSpec((tb, out_d), lambda i: (i, 0)),
        out_shape=jax.ShapeDtypeStruct((n, out_d), jnp.float32),
    )(yt, yt, g1, g2)
    return out


# speedup vs baseline: 1.0646x; 1.0011x over previous
import functools

import jax
import jax.numpy as jnp
import numpy as np
from jax import lax
from jax.experimental import pallas as pl
from jax.experimental.pallas import tpu as pltpu
from jax.experimental.pallas import tpu_sc as plsc

_EPS = float(np.finfo(np.float64).eps)


def _gating_body(x_ref, wg_ref, i1_ref, i2_ref, g1_ref, g2_ref, cnt_ref,
                 acc_ref):
    i = pl.program_id(0)
    logits = lax.dot_general(x_ref[...], wg_ref[...], (((1,), (0,)), ((), ())),
                             preferred_element_type=jnp.float32)
    e = logits.shape[1]
    col = lax.broadcasted_iota(jnp.int32, logits.shape, 1)
    m1 = jnp.max(logits, axis=1, keepdims=True)
    i1 = jnp.min(jnp.where(logits == m1, col, e), axis=1, keepdims=True)
    masked = jnp.where(col == i1, -jnp.inf, logits)
    m2 = jnp.max(masked, axis=1, keepdims=True)
    i2 = jnp.min(jnp.where(masked == m2, col, e), axis=1, keepdims=True)
    t = jnp.exp(m2 - m1)
    denom = 1.0 + t
    i1_ref[...] = i1
    i2_ref[...] = i2
    g1_ref[...] = 1.0 / denom
    g2_ref[...] = t / denom
    hist = (jnp.sum((i1 == col[: i1.shape[0]]).astype(jnp.int32), axis=0,
                    keepdims=True)
            + jnp.sum((i2 == col[: i2.shape[0]]).astype(jnp.int32), axis=0,
                      keepdims=True))

    @pl.when(i == 0)
    def _():
        acc_ref[...] = jnp.zeros_like(acc_ref)

    acc_ref[0:1, :] += hist

    @pl.when(i == pl.num_programs(0) - 1)
    def _():
        cnt_ref[...] = acc_ref[0:1, :]


def _routing_body(i1_ref, i2_ref, poffs_ref, p0_ref, p1_ref, acc_ref, lt_ref):
    i = pl.program_id(0)
    tr = i1_ref.shape[0]
    ee = jnp.concatenate([i1_ref[...], i2_ref[...]], axis=0)
    e = poffs_ref.shape[1] - 1
    col = lax.broadcasted_iota(jnp.int32, (2 * tr, e), 1)
    onehot_b = ee == col
    onehot_f = onehot_b.astype(jnp.float32)

    @pl.when(i == 0)
    def _():
        acc_ref[...] = jnp.zeros_like(acc_ref)
        r = lax.broadcasted_iota(jnp.int32, (2 * tr, 2 * tr), 0)
        c = lax.broadcasted_iota(jnp.int32, (2 * tr, 2 * tr), 1)
        lt_ref[...] = (c < r).astype(jnp.float32)

    excl = lax.dot_general(lt_ref[...], onehot_f, (((1,), (0,)), ((), ())),
                           preferred_element_type=jnp.float32)
    rank = jnp.sum(excl * onehot_f, axis=1, keepdims=True).astype(jnp.int32)
    base = jnp.sum(jnp.where(onehot_b, poffs_ref[0:1, :e], 0), axis=1,
                   keepdims=True)
    run = jnp.sum(jnp.where(onehot_b, acc_ref[0:1, :], 0), axis=1,
                  keepdims=True)
    slot = base + run + rank
    p0_ref[...] = slot[:tr]
    p1_ref[...] = slot[tr:]
    acc_ref[0:1, :] += jnp.sum(onehot_b.astype(jnp.int32), axis=0,
                               keepdims=True)


def _gmm_body(te_ref, na_ref, xs_ref, w1_ref, b1_ref, w2_ref, b2_ref, ys_ref):
    i = pl.program_id(0)

    @pl.when(i < na_ref[0])
    def _():
        h = lax.dot_general(xs_ref[...], w1_ref[0], (((1,), (0,)), ((), ())),
                            preferred_element_type=jnp.float32)
        h = jnp.maximum(h + b1_ref[0], 0.0)
        o = lax.dot_general(h, w2_ref[0], (((1,), (0,)), ((), ())),
                            preferred_element_type=jnp.float32)
        ys_ref[...] = jnp.exp(jnp.tanh(o + b2_ref[0]) * 10.0)


def _combine_body(y1_ref, y2_ref, g1_ref, g2_ref, o_ref):
    s = g1_ref[...] * y1_ref[...] + g2_ref[...] * y2_ref[...]
    o_ref[...] = jnp.log(jnp.where(s == 0.0, _EPS, s))


def _sc_scatter_rows(x, pos0, pos1, p):
    n, d = x.shape
    nw = 32
    ch = 128
    t_per_w = n // nw
    nch = t_per_w // ch
    mesh = plsc.VectorSubcoreMesh(core_axis_name="c", subcore_axis_name="s")

    @functools.partial(
        pl.kernel,
        mesh=mesh,
        out_type=jax.ShapeDtypeStruct((p, d), jnp.float32),
        scratch_types=[
            pltpu.VMEM((2, ch), jnp.int32),
            pltpu.VMEM((ch, d), jnp.float32),
            pltpu.SemaphoreType.DMA,
        ],
    )
    def scatter_k(x_hbm, p0_hbm, p1_hbm, xs_hbm, posb, rows, sem):
        wid = lax.axis_index("s") * 2 + lax.axis_index("c")
        for cc in range(nch):
            off = base = wid * t_per_w + cc * ch
            li0 = pltpu.async_copy(p0_hbm.at[pl.ds(off, ch)], posb.at[0], sem)
            li1 = pltpu.async_copy(p1_hbm.at[pl.ds(off, ch)], posb.at[1], sem)
            lr = pltpu.async_copy(x_hbm.at[pl.ds(off, ch)], rows, sem)
            li0.wait()
            li1.wait()
            lr.wait()
            cp0 = pltpu.async_copy(rows, xs_hbm.at[posb.at[0]], sem)
            cp1 = pltpu.async_copy(rows, xs_hbm.at[posb.at[1]], sem)
            cp0.wait()
            cp1.wait()

    return scatter_k(x, pos0, pos1)


def _sc_gather(table, idx):
    _, d = table.shape
    b = idx.shape[0]
    nw = 32
    ch = 128
    b_per_w = b // nw
    nch = b_per_w // ch
    mesh = plsc.VectorSubcoreMesh(core_axis_name="c", subcore_axis_name="s")

    @functools.partial(
        pl.kernel,
        mesh=mesh,
        out_type=jax.ShapeDtypeStruct((b, d), jnp.float32),
        scratch_types=[
            pltpu.VMEM((nch, ch), jnp.int32),
            pltpu.VMEM((ch, d), jnp.float32),
            pltpu.SemaphoreType.DMA,
        ],
    )
    def gather_k(table_hbm, idx_hbm, out_hbm, idxb, rows, sem):
        wid = lax.axis_index("s") * 2 + lax.axis_index("c")
        base = wid * b_per_w
        for j in range(nch):
            pltpu.sync_copy(idx_hbm.at[pl.ds(base + j * ch, ch)], idxb.at[j])
        for j in range(nch):
            pltpu.async_copy(table_hbm.at[idxb.at[j]], rows, sem).wait()
            pltpu.sync_copy(rows, out_hbm.at[pl.ds(base + j * ch, ch)])

    return gather_k(table, idx)


def kernel(x, w_gate, W1, b1, W2, b2):
    n, d = x.shape
    e = w_gate.shape[1]
    h = W1.shape[2]
    out_d = W2.shape[2]
    tm = 192
    p = 2 * n + (tm - 1) * min(e, 2 * n)
    p = ((p + tm - 1) // tm) * tm
    tn = p // tm

    tg = 1024
    i1, i2, g1, g2, counts = pl.pallas_call(
        _gating_body,
        grid=(n // tg,),
        in_specs=[
            pl.BlockSpec((tg, d), lambda i: (i, 0)),
            pl.BlockSpec((d, e), lambda i: (0, 0)),
        ],
        out_specs=[
            pl.BlockSpec((tg, 1), lambda i: (i, 0)),
            pl.BlockSpec((tg, 1), lambda i: (i, 0)),
            pl.BlockSpec((tg, 1), lambda i: (i, 0)),
            pl.BlockSpec((tg, 1), lambda i: (i, 0)),
            pl.BlockSpec((1, e), lambda i: (0, 0)),
        ],
        out_shape=[
            jax.ShapeDtypeStruct((n, 1), jnp.int32),
            jax.ShapeDtypeStruct((n, 1), jnp.int32),
            jax.ShapeDtypeStruct((n, 1), jnp.float32),
            jax.ShapeDtypeStruct((n, 1), jnp.float32),
            jax.ShapeDtypeStruct((1, e), jnp.int32),
        ],
        scratch_shapes=[pltpu.VMEM((8, e), jnp.int32)],
    )(x, w_gate)

    pcounts = ((counts[0] + tm - 1) // tm) * tm
    poffs = jnp.concatenate(
        [jnp.zeros((1,), jnp.int32), jnp.cumsum(pcounts).astype(jnp.int32)])
    tile_start = jnp.arange(tn, dtype=jnp.int32) * tm
    tile_expert = jnp.minimum(
        jnp.sum((tile_start[:, None] >= poffs[None, 1:]).astype(jnp.int32),
                axis=1), e - 1).astype(jnp.int32)
    n_active = (poffs[e] // tm).reshape(1)

    tr = 512
    pos0, pos1 = pl.pallas_call(
        _routing_body,
        grid=(n // tr,),
        in_specs=[
            pl.BlockSpec((tr, 1), lambda i: (i, 0)),
            pl.BlockSpec((tr, 1), lambda i: (i, 0)),
            pl.BlockSpec((1, e + 1), lambda i: (0, 0)),
        ],
        out_specs=[
            pl.BlockSpec((tr, 1), lambda i: (i, 0)),
            pl.BlockSpec((tr, 1), lambda i: (i, 0)),
        ],
        out_shape=[
            jax.ShapeDtypeStruct((n, 1), jnp.int32),
            jax.ShapeDtypeStruct((n, 1), jnp.int32),
        ],
        scratch_shapes=[pltpu.VMEM((8, e), jnp.int32),
                        pltpu.VMEM((2 * tr, 2 * tr), jnp.float32)],
    )(i1, i2, poffs.reshape(1, e + 1))
    pos0 = pos0.reshape(n)
    pos1 = pos1.reshape(n)

    xs = _sc_scatter_rows(x, pos0, pos1, p)

    grid_spec = pltpu.PrefetchScalarGridSpec(
        num_scalar_prefetch=2,
        grid=(tn,),
        in_specs=[
            pl.BlockSpec((tm, d),
                         lambda i, te, na: (jnp.minimum(i, na[0] - 1), 0)),
            pl.BlockSpec((1, d, h), lambda i, te, na: (te[i], 0, 0)),
            pl.BlockSpec((1, 1, h), lambda i, te, na: (te[i], 0, 0)),
            pl.BlockSpec((1, h, out_d), lambda i, te, na: (te[i], 0, 0)),
            pl.BlockSpec((1, 1, out_d), lambda i, te, na: (te[i], 0, 0)),
        ],
        out_specs=pl.BlockSpec(
            (tm, out_d), lambda i, te, na: (jnp.minimum(i, na[0] - 1), 0)),
    )
    ys = pl.pallas_call(
        _gmm_body,
        grid_spec=grid_spec,
        out_shape=jax.ShapeDtypeStruct((p, out_d), jnp.float32),
    )(tile_expert, n_active, xs, W1, b1.reshape(e, 1, h), W2,
      b2.reshape(e, 1, out_d))

    yt = _sc_gather(ys, jnp.concatenate([pos0, pos1]))

    tb = 1024
    out = pl.pallas_call(
        _combine_body,
        grid=(n // tb,),
        in_specs=[
            pl.BlockSpec((tb, out_d), lambda i: (i, 0)),
            pl.BlockSpec((tb, out_d), lambda i: (i + n // tb, 0)),
            pl.BlockSpec((tb, 1), lambda i: (i, 0)),
            pl.BlockSpec((tb, 1), lambda i: (i, 0)),
        ],
        out_specs=pl.BlockSpec((tb, out_d), lambda i: (i, 0)),
        out_shape=jax.ShapeDtypeStruct((n, out_d), jnp.float32),
    )(yt, yt, g1, g2)
    return out
